# baseline (speedup 1.0000x reference)
import functools

import jax
import jax.numpy as jnp
from jax import lax
from jax.experimental import pallas as pl
from jax.experimental.pallas import tpu as pltpu
from jax.experimental.pallas import tpu_sc as plsc

VOCAB = 1000000
EMBED = 64
BATCH = 16384
LANES = 16
ROWPAIR = 2 * EMBED
IDX_CHUNK = 128

try:
    _info = plsc.get_sparse_core_info()
    _NC, _NS = _info.num_cores, _info.num_subcores
except Exception:
    _NC, _NS = 2, 16
_NW = _NC * _NS
_BPW = BATCH // _NW
_HALF = _BPW // 2

_mesh = plsc.VectorSubcoreMesh(core_axis_name="c", subcore_axis_name="s")


def _build_gather_kernel():
    @functools.partial(
        pl.kernel,
        mesh=_mesh,
        out_type=jax.ShapeDtypeStruct((BATCH, ROWPAIR), jnp.float32),
        scratch_types=[
            pltpu.VMEM((_BPW,), jnp.int32),
            pltpu.VMEM((_BPW,), jnp.int32),
            pltpu.VMEM((_BPW, ROWPAIR), jnp.float32),
            pltpu.SemaphoreType.DMA,
        ],
        compiler_params=pltpu.CompilerParams(use_tc_tiling_on_sc=True),
    )
    def gather_kernel(idx_hbm, tab_hbm, out_hbm, idx_v, pair_v, rows_v, sem):
        wid = lax.axis_index("s") * _NC + lax.axis_index("c")
        base = wid * _BPW
        pltpu.sync_copy(idx_hbm.at[pl.ds(base, _BPW)], idx_v)

        def halve(g, carry):
            v = idx_v[pl.ds(g * LANES, LANES)]
            pair_v[pl.ds(g * LANES, LANES)] = jnp.right_shift(v, 1)
            return carry

        lax.fori_loop(0, _BPW // LANES, halve, 0)

        copies = []
        for k in range(_BPW // IDX_CHUNK):
            sl = pl.ds(k * IDX_CHUNK, IDX_CHUNK)
            copies.append(pltpu.async_copy(
                tab_hbm.at[pair_v.at[sl]], rows_v.at[sl], sem))
        for cp in copies:
            cp.wait()
        pltpu.sync_copy(rows_v, out_hbm.at[pl.ds(base, _BPW)])

    return gather_kernel


def _build_dot_kernel():
    @functools.partial(
        pl.kernel,
        mesh=_mesh,
        out_type=jax.ShapeDtypeStruct((BATCH,), jnp.float32),
        scratch_types=[
            pltpu.VMEM((_BPW,), jnp.int32),
            pltpu.VMEM((_BPW,), jnp.int32),
            pltpu.VMEM((_HALF, ROWPAIR), jnp.float32),
            pltpu.VMEM((_HALF, ROWPAIR), jnp.float32),
            pltpu.VMEM((_BPW,), jnp.float32),
        ],
        compiler_params=pltpu.CompilerParams(use_tc_tiling_on_sc=False),
    )
    def dot_kernel(cidx_hbm, xidx_hbm, crows_hbm, xrows_hbm, out_hbm,
                   cidx_v, xidx_v, crows_v, xrows_v, score_v):
        wid = lax.axis_index("s") * _NC + lax.axis_index("c")
        base = wid * _BPW
        pltpu.sync_copy(cidx_hbm.at[pl.ds(base, _BPW)], cidx_v)
        pltpu.sync_copy(xidx_hbm.at[pl.ds(base, _BPW)], xidx_v)

        lane = lax.iota(jnp.int32, LANES)
        dnums = lax.GatherDimensionNumbers(
            offset_dims=(), collapsed_slice_dims=(0,), start_index_map=(0,))

        def hsum(vec):
            for s in (1, 2, 4, 8):
                perm = lane ^ s
                vec = vec + lax.gather(
                    vec, perm[:, None], dnums, (1,),
                    mode=lax.GatherScatterMode.PROMISE_IN_BOUNDS)
            return vec

        for h in range(2):
            hbase = h * _HALF
            pltpu.sync_copy(crows_hbm.at[pl.ds(base + hbase, _HALF)], crows_v)
            pltpu.sync_copy(xrows_hbm.at[pl.ds(base + hbase, _HALF)], xrows_v)

            def group_body(g, carry):
                cpar = jnp.bitwise_and(
                    cidx_v[pl.ds(hbase + g * LANES, LANES)], 1) * EMBED
                xpar = jnp.bitwise_and(
                    xidx_v[pl.ds(hbase + g * LANES, LANES)], 1) * EMBED
                scores = jnp.zeros((LANES,), jnp.float32)
                for r16 in range(LANES):
                    r = g * LANES + r16
                    coff = cpar[r16]
                    xoff = xpar[r16]
                    acc = jnp.zeros((LANES,), jnp.float32)
                    for j in range(EMBED // LANES):
                        acc = acc + (
                            crows_v[r, pl.ds(coff + j * LANES, LANES)]
                            * xrows_v[r, pl.ds(xoff + j * LANES, LANES)])
                    scores = jnp.where(lane == r16, hsum(acc), scores)
                score_v[pl.ds(hbase + g * LANES, LANES)] = scores
                return carry

            lax.fori_loop(0, _HALF // LANES, group_body, 0)

        pltpu.sync_copy(score_v, out_hbm.at[pl.ds(base, _BPW)])

    return dot_kernel


_gather_kernel = _build_gather_kernel()
_dot_kernel = _build_dot_kernel()


def kernel(center, context, center_table, context_table):
    center = center.astype(jnp.int32)
    context = context.astype(jnp.int32)
    crows = _gather_kernel(center, center_table.reshape(VOCAB // 2, ROWPAIR))
    xrows = _gather_kernel(context, context_table.reshape(VOCAB // 2, ROWPAIR))
    return _dot_kernel(center, context, crows, xrows)


# speedup vs baseline: 1.0174x; 1.0174x over previous
import functools

import jax
import jax.numpy as jnp
from jax import lax
from jax.experimental import pallas as pl
from jax.experimental.pallas import tpu as pltpu
from jax.experimental.pallas import tpu_sc as plsc

VOCAB = 1000000
EMBED = 64
BATCH = 16384
LANES = 16
IDX_CHUNK = 128

try:
    _info = plsc.get_sparse_core_info()
    _NC, _NS = _info.num_cores, _info.num_subcores
except Exception:
    _NC, _NS = 2, 16
_NW = _NC * _NS
_BPW = BATCH // _NW


def _build_sc_kernel():
    mesh = plsc.VectorSubcoreMesh(core_axis_name="c", subcore_axis_name="s")

    @functools.partial(
        pl.kernel,
        mesh=mesh,
        out_type=jax.ShapeDtypeStruct((BATCH,), jnp.float32),
        scratch_types=[
            pltpu.VMEM((_BPW,), jnp.int32),
            pltpu.VMEM((_BPW,), jnp.int32),
            pltpu.VMEM((_BPW, EMBED), jnp.float32),
            pltpu.VMEM((_BPW, EMBED), jnp.float32),
            pltpu.VMEM((_BPW,), jnp.float32),
            pltpu.SemaphoreType.DMA,
        ],
        compiler_params=pltpu.CompilerParams(use_tc_tiling_on_sc=False),
    )
    def sc_kernel(center_hbm, context_hbm, ctab_hbm, xtab_hbm, out_hbm,
                  cidx_v, xidx_v, crows_v, xrows_v, score_v, sem):
        wid = lax.axis_index("s") * _NC + lax.axis_index("c")
        base = wid * _BPW

        pltpu.sync_copy(center_hbm.at[pl.ds(base, _BPW)], cidx_v)
        pltpu.sync_copy(context_hbm.at[pl.ds(base, _BPW)], xidx_v)

        copies = []
        for k in range(_BPW // IDX_CHUNK):
            sl = pl.ds(k * IDX_CHUNK, IDX_CHUNK)
            copies.append(pltpu.async_copy(
                ctab_hbm.at[cidx_v.at[sl]], crows_v.at[sl], sem))
            copies.append(pltpu.async_copy(
                xtab_hbm.at[xidx_v.at[sl]], xrows_v.at[sl], sem))
        for cp in copies:
            cp.wait()

        lane = lax.iota(jnp.int32, LANES)
        dnums = lax.GatherDimensionNumbers(
            offset_dims=(), collapsed_slice_dims=(0,), start_index_map=(0,))

        def hsum(vec):
            for s in (1, 2, 4, 8):
                perm = lane ^ s
                vec = vec + lax.gather(
                    vec, perm[:, None], dnums, (1,),
                    mode=lax.GatherScatterMode.PROMISE_IN_BOUNDS)
            return vec

        def group_body(g, carry):
            scores = jnp.zeros((LANES,), jnp.float32)
            for r16 in range(LANES):
                r = g * LANES + r16
                acc = crows_v[r, pl.ds(0, LANES)] * xrows_v[r, pl.ds(0, LANES)]
                for j in range(1, EMBED // LANES):
                    acc = acc + (crows_v[r, pl.ds(j * LANES, LANES)]
                                 * xrows_v[r, pl.ds(j * LANES, LANES)])
                scores = jnp.where(lane == r16, hsum(acc), scores)
            score_v[pl.ds(g * LANES, LANES)] = scores
            return carry

        lax.fori_loop(0, _BPW // LANES, group_body, 0)

        pltpu.sync_copy(score_v, out_hbm.at[pl.ds(base, _BPW)])

    return sc_kernel


_sc_kernel = _build_sc_kernel()


def kernel(center, context, center_table, context_table):
    return _sc_kernel(center.astype(jnp.int32), context.astype(jnp.int32),
                      center_table, context_table)
